# Initial kernel scaffold; baseline (speedup 1.0000x reference)
import jax
import jax.numpy as jnp
from jax.experimental import pallas as pl


def kernel(features, edge_index, W0, b0, W1, b1):
    raise NotImplementedError("write your pallas kernel here")


# speedup vs baseline: 3.1483x; 3.1483x over previous
import functools

import jax
import jax.numpy as jnp
from jax import lax
from jax.experimental import pallas as pl
from jax.experimental.pallas import tpu as pltpu
from jax.experimental.pallas import tpu_sc as plsc

N = 100000
E = 1600000
D = 40
ALPHA = 0.1
K = 10

HALF = 50000
NT = 16
HROWS = 50048
PN = 2 * HROWS
PAD = HROWS - HALF
TRASH = 50016
ORPT = HROWS // NT
EBLK = 128
NBLK = 782
EPT = NBLK * EBLK
EP = NT * EPT


def _prop_body(g_hbm, src_hbm, dst_hbm, zeros_hbm, out_hbm,
               agg, sidx, didx, lidx, rows, sem):
    c = lax.axis_index("c")
    s = lax.axis_index("s")
    pltpu.sync_copy(zeros_hbm, agg.at[pl.ds(s * ORPT, ORPT)])
    plsc.subcore_barrier()

    dbase = (c * HALF).astype(jnp.int32)

    def eblock(j, carry):
        off = s * EPT + j * EBLK
        pltpu.sync_copy(src_hbm.at[pl.ds(off, EBLK)], sidx)
        pltpu.sync_copy(dst_hbm.at[pl.ds(off, EBLK)], didx)
        for k in range(EBLK // 16):
            sl = pl.ds(k * 16, 16)
            sv = sidx[sl]
            sidx[sl] = sv + jnp.where(sv >= HALF, jnp.int32(PAD), jnp.int32(0))
            dv = didx[sl] - dbase
            ok = (dv >= 0) & (dv < HALF)
            lidx[sl] = jnp.where(ok, dv, jnp.int32(TRASH))
        pltpu.async_copy(g_hbm.at[sidx], rows, sem).wait()
        pltpu.sync_copy(rows, agg.at[lidx], add=True)
        return carry

    lax.fori_loop(0, NBLK, eblock, 0)
    plsc.subcore_barrier()
    pltpu.sync_copy(agg.at[pl.ds(s * ORPT, ORPT)],
                    out_hbm.at[pl.ds(c * HROWS + s * ORPT, ORPT)])


@functools.cache
def _build_prop():
    return pl.kernel(
        _prop_body,
        out_type=jax.ShapeDtypeStruct((PN, D), jnp.float32),
        mesh=plsc.VectorSubcoreMesh(core_axis_name="c", subcore_axis_name="s"),
        scratch_types=[
            pltpu.VMEM_SHARED((HROWS, D), jnp.float32),
            pltpu.VMEM((EBLK,), jnp.int32),
            pltpu.VMEM((EBLK,), jnp.int32),
            pltpu.VMEM((EBLK,), jnp.int32),
            pltpu.VMEM((EBLK, D), jnp.float32),
            pltpu.SemaphoreType.DMA,
        ],
        compiler_params=pltpu.CompilerParams(use_tc_tiling_on_sc=False),
    )


def _prop(g, srcp, dstp, zeros_blk):
    return _build_prop()(g, srcp, dstp, zeros_blk)


MBLK = 1000


def _mlp_body(f_ref, w0_ref, b0_ref, w1_ref, b1_ref, deg_ref,
              h0_ref, g0_ref, sag_ref, sah_ref):
    f = f_ref[...]
    h = jnp.maximum(
        jnp.dot(f, w0_ref[...], preferred_element_type=jnp.float32)
        + b0_ref[...], 0.0)
    h0 = jnp.dot(h, w1_ref[...], preferred_element_type=jnp.float32) + b1_ref[...]
    deg = deg_ref[...][:, 0:1]
    norm = jnp.where(deg > 0, lax.rsqrt(jnp.maximum(deg, 1.0)), 0.0)
    h0_ref[...] = h0
    g0_ref[...] = h0 * norm
    sag_ref[...] = jnp.broadcast_to((1.0 - ALPHA) * norm * norm, h0.shape)
    sah_ref[...] = jnp.broadcast_to((1.0 - ALPHA) * norm, h0.shape)


def _mlp(features, W0, b0, W1, b1, deg40):
    out = jax.ShapeDtypeStruct((N, D), jnp.float32)
    return pl.pallas_call(
        _mlp_body,
        grid=(N // MBLK,),
        in_specs=[
            pl.BlockSpec((MBLK, 128), lambda i: (i, 0)),
            pl.BlockSpec((128, 64), lambda i: (0, 0)),
            pl.BlockSpec((1, 64), lambda i: (0, 0)),
            pl.BlockSpec((64, D), lambda i: (0, 0)),
            pl.BlockSpec((1, D), lambda i: (0, 0)),
            pl.BlockSpec((MBLK, D), lambda i: (i, 0)),
        ],
        out_specs=[pl.BlockSpec((MBLK, D), lambda i: (i, 0))] * 4,
        out_shape=[out, out, out, out],
    )(features, W0, b0, W1, b1, deg40)


UBLK = 256


def _upd_body(agg_ref, sa_ref, base_ref, out_ref):
    out_ref[...] = sa_ref[...] * agg_ref[...] + ALPHA * base_ref[...]


def _upd(agg, sa, base):
    return pl.pallas_call(
        _upd_body,
        grid=(PN // UBLK,),
        in_specs=[pl.BlockSpec((UBLK, D), lambda i: (i, 0))] * 3,
        out_specs=pl.BlockSpec((UBLK, D), lambda i: (i, 0)),
        out_shape=jax.ShapeDtypeStruct((PN, D), jnp.float32),
    )(agg, sa, base)


def _pad_rows(x):
    return jnp.pad(x.reshape(2, HALF, D), ((0, 0), (0, PAD), (0, 0))).reshape(PN, D)


def _unpad_rows(x):
    return x.reshape(2, HROWS, D)[:, :HALF].reshape(N, D)


def kernel(features, edge_index, W0, b0, W1, b1):
    src = edge_index[0].astype(jnp.int32)
    dst = edge_index[1].astype(jnp.int32)
    srcp = jnp.concatenate([src, jnp.zeros((EP - E,), jnp.int32)])
    dstp = jnp.concatenate([dst, jnp.full((EP - E,), -1, jnp.int32)])
    zeros_blk = jnp.zeros((ORPT, D), jnp.float32)

    ones_t = jnp.ones((PN, D), jnp.float32)
    deg40 = _unpad_rows(_prop(ones_t, srcp, dstp, zeros_blk))

    h0, g0, sag, sah = _mlp(features, W0, b0.reshape(1, -1), W1,
                            b1.reshape(1, -1), deg40)
    h0p = _pad_rows(h0)
    g0p = _pad_rows(g0)
    sagp = _pad_rows(sag)
    sahp = _pad_rows(sah)

    g = g0p
    for _ in range(K - 1):
        u = _prop(g, srcp, dstp, zeros_blk)
        g = _upd(u, sagp, g0p)
    u = _prop(g, srcp, dstp, zeros_blk)
    return _unpad_rows(_upd(u, sahp, h0p))


# speedup vs baseline: 3.6589x; 1.1622x over previous
import functools

import jax
import jax.numpy as jnp
from jax import lax
from jax.experimental import pallas as pl
from jax.experimental.pallas import tpu as pltpu
from jax.experimental.pallas import tpu_sc as plsc

N = 100000
E = 1600000
D = 40
ALPHA = 0.1
K = 10

HALF = 50000
NT = 16
HROWS = 50048
PN = 2 * HROWS
PAD = HROWS - HALF
TRASH = 50016
ORPT = HROWS // NT
EBLK = 96
SBI = 4
NSUP = 261
NBLK = NSUP * SBI
EPT = NBLK * EBLK
EP = NT * EPT
NROW = EP // EBLK


def _prop_body(g_hbm, src_hbm, dstl_hbm, zeros_hbm, out_hbm,
               agg, sidx, didx, rows, gsem):
    c = lax.axis_index("c")
    s = lax.axis_index("s")
    pltpu.sync_copy(zeros_hbm, agg.at[pl.ds(s * ORPT, ORPT)])
    plsc.subcore_barrier()

    def sup(u, carry):
        rowbase = s * NBLK + u * SBI
        pltpu.sync_copy(src_hbm.at[pl.ds(rowbase, SBI)], sidx)
        pltpu.sync_copy(dstl_hbm.at[pl.ds(c * NROW + rowbase, SBI)], didx)
        for k in range(SBI):
            pltpu.async_copy(g_hbm.at[sidx.at[k]], rows, gsem).wait()
            pltpu.sync_copy(rows, agg.at[didx.at[k]], add=True)
        return carry

    lax.fori_loop(0, NSUP, sup, 0)

    plsc.subcore_barrier()
    pltpu.sync_copy(agg.at[pl.ds(s * ORPT, ORPT)],
                    out_hbm.at[pl.ds(c * HROWS + s * ORPT, ORPT)])


@functools.cache
def _build_prop():
    return pl.kernel(
        _prop_body,
        out_type=jax.ShapeDtypeStruct((PN, D), jnp.float32),
        mesh=plsc.VectorSubcoreMesh(core_axis_name="c", subcore_axis_name="s"),
        scratch_types=[
            pltpu.VMEM_SHARED((HROWS, D), jnp.float32),
            pltpu.VMEM((SBI, EBLK), jnp.int32),
            pltpu.VMEM((SBI, EBLK), jnp.int32),
            pltpu.VMEM((EBLK, D), jnp.float32),
            pltpu.SemaphoreType.DMA,
        ],
        compiler_params=pltpu.CompilerParams(use_tc_tiling_on_sc=False),
    )


def _prop(g, srcr, dstl, zeros_blk):
    return _build_prop()(g, srcr, dstl, zeros_blk)


MBLK = 1000


def _mlp_body(f_ref, w0_ref, b0_ref, w1_ref, b1_ref, deg_ref,
              h0_ref, g0_ref, sag_ref, sah_ref):
    f = f_ref[...]
    h = jnp.maximum(
        jnp.dot(f, w0_ref[...], preferred_element_type=jnp.float32)
        + b0_ref[...], 0.0)
    h0 = jnp.dot(h, w1_ref[...], preferred_element_type=jnp.float32) + b1_ref[...]
    deg = deg_ref[...][:, 0:1]
    norm = jnp.where(deg > 0, lax.rsqrt(jnp.maximum(deg, 1.0)), 0.0)
    h0_ref[...] = h0
    g0_ref[...] = h0 * norm
    sag_ref[...] = jnp.broadcast_to((1.0 - ALPHA) * norm * norm, h0.shape)
    sah_ref[...] = jnp.broadcast_to((1.0 - ALPHA) * norm, h0.shape)


def _mlp(features, W0, b0, W1, b1, deg40):
    out = jax.ShapeDtypeStruct((N, D), jnp.float32)
    return pl.pallas_call(
        _mlp_body,
        grid=(N // MBLK,),
        in_specs=[
            pl.BlockSpec((MBLK, 128), lambda i: (i, 0)),
            pl.BlockSpec((128, 64), lambda i: (0, 0)),
            pl.BlockSpec((1, 64), lambda i: (0, 0)),
            pl.BlockSpec((64, D), lambda i: (0, 0)),
            pl.BlockSpec((1, D), lambda i: (0, 0)),
            pl.BlockSpec((MBLK, D), lambda i: (i, 0)),
        ],
        out_specs=[pl.BlockSpec((MBLK, D), lambda i: (i, 0))] * 4,
        out_shape=[out, out, out, out],
    )(features, W0, b0, W1, b1, deg40)


UBLK = 256


def _upd_body(agg_ref, sa_ref, base_ref, out_ref):
    out_ref[...] = sa_ref[...] * agg_ref[...] + ALPHA * base_ref[...]


def _upd(agg, sa, base):
    return pl.pallas_call(
        _upd_body,
        grid=(PN // UBLK,),
        in_specs=[pl.BlockSpec((UBLK, D), lambda i: (i, 0))] * 3,
        out_specs=pl.BlockSpec((UBLK, D), lambda i: (i, 0)),
        out_shape=jax.ShapeDtypeStruct((PN, D), jnp.float32),
    )(agg, sa, base)


def _pad_rows(x):
    return jnp.pad(x.reshape(2, HALF, D), ((0, 0), (0, PAD), (0, 0))).reshape(PN, D)


def _unpad_rows(x):
    return x.reshape(2, HROWS, D)[:, :HALF].reshape(N, D)


def kernel(features, edge_index, W0, b0, W1, b1):
    src = edge_index[0].astype(jnp.int32)
    dst = edge_index[1].astype(jnp.int32)
    srcp = jnp.concatenate([src, jnp.zeros((EP - E,), jnp.int32)])
    srcr = jnp.where(srcp >= HALF, srcp + PAD, srcp).reshape(NROW, EBLK)
    dstp = jnp.concatenate([dst, jnp.full((EP - E,), -1, jnp.int32)])
    halves = []
    for cc in (0, 1):
        l = dstp - cc * HALF
        halves.append(jnp.where((l >= 0) & (l < HALF), l, TRASH))
    dstl = jnp.concatenate(halves).reshape(2 * NROW, EBLK)
    zeros_blk = jnp.zeros((ORPT, D), jnp.float32)

    ones_t = jnp.ones((PN, D), jnp.float32)
    deg40 = _unpad_rows(_prop(ones_t, srcr, dstl, zeros_blk))

    h0, g0, sag, sah = _mlp(features, W0, b0.reshape(1, -1), W1,
                            b1.reshape(1, -1), deg40)
    h0p = _pad_rows(h0)
    g0p = _pad_rows(g0)
    sagp = _pad_rows(sag)
    sahp = _pad_rows(sah)

    g = g0p
    for _ in range(K - 1):
        u = _prop(g, srcr, dstl, zeros_blk)
        g = _upd(u, sagp, g0p)
    u = _prop(g, srcr, dstl, zeros_blk)
    return _unpad_rows(_upd(u, sahp, h0p))


# speedup vs baseline: 3.9950x; 1.0919x over previous
import functools

import jax
import jax.numpy as jnp
from jax import lax
from jax.experimental import pallas as pl
from jax.experimental.pallas import tpu as pltpu
from jax.experimental.pallas import tpu_sc as plsc

N = 100000
E = 1600000
D = 40
ALPHA = 0.1
K = 10

HALF = 50000
NT = 16
HROWS = 50048
PN = 2 * HROWS
PAD = HROWS - HALF
TRASH = 50016
ORPT = HROWS // NT
EBLK = 48
SBI = 8
NSUP = 261
NBLK = NSUP * SBI
EPT = NBLK * EBLK
EP = NT * EPT
NROW = EP // EBLK


def _prop_body(g_hbm, src_hbm, dstl_hbm, zeros_hbm, out_hbm,
               agg, sidx, didx, rows0, rows1, gsem0, gsem1):
    c = lax.axis_index("c")
    s = lax.axis_index("s")
    pltpu.sync_copy(zeros_hbm, agg.at[pl.ds(s * ORPT, ORPT)])
    plsc.subcore_barrier()

    rows = (rows0, rows1)
    gsem = (gsem0, gsem1)

    def sup(u, carry):
        rowbase = s * NBLK + u * SBI
        pltpu.sync_copy(src_hbm.at[pl.ds(rowbase, SBI)], sidx)
        pltpu.sync_copy(dstl_hbm.at[pl.ds(c * NROW + rowbase, SBI)], didx)
        gh = {}
        gh[0] = pltpu.async_copy(g_hbm.at[sidx.at[0]], rows[0], gsem[0])
        for k in range(SBI):
            if k + 1 < SBI:
                b = (k + 1) % 2
                gh[k + 1] = pltpu.async_copy(g_hbm.at[sidx.at[k + 1]],
                                             rows[b], gsem[b])
            gh[k].wait()
            pltpu.sync_copy(rows[k % 2], agg.at[didx.at[k]], add=True)
        return carry

    lax.fori_loop(0, NSUP, sup, 0)

    plsc.subcore_barrier()
    pltpu.sync_copy(agg.at[pl.ds(s * ORPT, ORPT)],
                    out_hbm.at[pl.ds(c * HROWS + s * ORPT, ORPT)])


@functools.cache
def _build_prop():
    return pl.kernel(
        _prop_body,
        out_type=jax.ShapeDtypeStruct((PN, D), jnp.float32),
        mesh=plsc.VectorSubcoreMesh(core_axis_name="c", subcore_axis_name="s"),
        scratch_types=[
            pltpu.VMEM_SHARED((HROWS, D), jnp.float32),
            pltpu.VMEM((SBI, EBLK), jnp.int32),
            pltpu.VMEM((SBI, EBLK), jnp.int32),
            pltpu.VMEM((EBLK, D), jnp.float32),
            pltpu.VMEM((EBLK, D), jnp.float32),
            pltpu.SemaphoreType.DMA,
            pltpu.SemaphoreType.DMA,
        ],
        compiler_params=pltpu.CompilerParams(use_tc_tiling_on_sc=False),
    )


def _prop(g, srcr, dstl, zeros_blk):
    return _build_prop()(g, srcr, dstl, zeros_blk)


MBLK = 1000


def _mlp_body(f_ref, w0_ref, b0_ref, w1_ref, b1_ref, deg_ref,
              h0_ref, g0_ref, sag_ref, sah_ref):
    f = f_ref[...]
    h = jnp.maximum(
        jnp.dot(f, w0_ref[...], preferred_element_type=jnp.float32)
        + b0_ref[...], 0.0)
    h0 = jnp.dot(h, w1_ref[...], preferred_element_type=jnp.float32) + b1_ref[...]
    deg = deg_ref[...][:, 0:1]
    norm = jnp.where(deg > 0, lax.rsqrt(jnp.maximum(deg, 1.0)), 0.0)
    h0_ref[...] = h0
    g0_ref[...] = h0 * norm
    sag_ref[...] = jnp.broadcast_to((1.0 - ALPHA) * norm * norm, h0.shape)
    sah_ref[...] = jnp.broadcast_to((1.0 - ALPHA) * norm, h0.shape)


def _mlp(features, W0, b0, W1, b1, deg40):
    out = jax.ShapeDtypeStruct((N, D), jnp.float32)
    return pl.pallas_call(
        _mlp_body,
        grid=(N // MBLK,),
        in_specs=[
            pl.BlockSpec((MBLK, 128), lambda i: (i, 0)),
            pl.BlockSpec((128, 64), lambda i: (0, 0)),
            pl.BlockSpec((1, 64), lambda i: (0, 0)),
            pl.BlockSpec((64, D), lambda i: (0, 0)),
            pl.BlockSpec((1, D), lambda i: (0, 0)),
            pl.BlockSpec((MBLK, D), lambda i: (i, 0)),
        ],
        out_specs=[pl.BlockSpec((MBLK, D), lambda i: (i, 0))] * 4,
        out_shape=[out, out, out, out],
    )(features, W0, b0, W1, b1, deg40)


UBLK = 256


def _upd_body(agg_ref, sa_ref, base_ref, out_ref):
    out_ref[...] = sa_ref[...] * agg_ref[...] + ALPHA * base_ref[...]


def _upd(agg, sa, base):
    return pl.pallas_call(
        _upd_body,
        grid=(PN // UBLK,),
        in_specs=[pl.BlockSpec((UBLK, D), lambda i: (i, 0))] * 3,
        out_specs=pl.BlockSpec((UBLK, D), lambda i: (i, 0)),
        out_shape=jax.ShapeDtypeStruct((PN, D), jnp.float32),
    )(agg, sa, base)


def _pad_rows(x):
    return jnp.pad(x.reshape(2, HALF, D), ((0, 0), (0, PAD), (0, 0))).reshape(PN, D)


def _unpad_rows(x):
    return x.reshape(2, HROWS, D)[:, :HALF].reshape(N, D)


def kernel(features, edge_index, W0, b0, W1, b1):
    src = edge_index[0].astype(jnp.int32)
    dst = edge_index[1].astype(jnp.int32)
    srcp = jnp.concatenate([src, jnp.zeros((EP - E,), jnp.int32)])
    srcr = jnp.where(srcp >= HALF, srcp + PAD, srcp).reshape(NROW, EBLK)
    dstp = jnp.concatenate([dst, jnp.full((EP - E,), -1, jnp.int32)])
    halves = []
    for cc in (0, 1):
        l = dstp - cc * HALF
        halves.append(jnp.where((l >= 0) & (l < HALF), l, TRASH))
    dstl = jnp.concatenate(halves).reshape(2 * NROW, EBLK)
    zeros_blk = jnp.zeros((ORPT, D), jnp.float32)

    ones_t = jnp.ones((PN, D), jnp.float32)
    deg40 = _unpad_rows(_prop(ones_t, srcr, dstl, zeros_blk))

    h0, g0, sag, sah = _mlp(features, W0, b0.reshape(1, -1), W1,
                            b1.reshape(1, -1), deg40)
    h0p = _pad_rows(h0)
    g0p = _pad_rows(g0)
    sagp = _pad_rows(sag)
    sahp = _pad_rows(sah)

    g = g0p
    for _ in range(K - 1):
        u = _prop(g, srcr, dstl, zeros_blk)
        g = _upd(u, sagp, g0p)
    u = _prop(g, srcr, dstl, zeros_blk)
    return _unpad_rows(_upd(u, sahp, h0p))


# speedup vs baseline: 4.0006x; 1.0014x over previous
import functools

import jax
import jax.numpy as jnp
from jax import lax
from jax.experimental import pallas as pl
from jax.experimental.pallas import tpu as pltpu
from jax.experimental.pallas import tpu_sc as plsc

N = 100000
E = 1600000
D = 40
ALPHA = 0.1
K = 10

HALF = 50000
NT = 16
HROWS = 50048
PN = 2 * HROWS
PAD = HROWS - HALF
TRASH = 50016
ORPT = HROWS // NT
EBLK = 48
SBI = 8
NSUP = 261
NBLK = NSUP * SBI
EPT = NBLK * EBLK
EP = NT * EPT
NROW = EP // EBLK


def _prop_body(g_hbm, src_hbm, dstl_hbm, zeros_hbm, out_hbm,
               agg, sidx, didx, rows0, rows1, gsem0, gsem1, ssem0, ssem1):
    c = lax.axis_index("c")
    s = lax.axis_index("s")
    pltpu.sync_copy(zeros_hbm, agg.at[pl.ds(s * ORPT, ORPT)])
    plsc.subcore_barrier()

    rows = (rows0, rows1)
    gsem = (gsem0, gsem1)
    ssem = (ssem0, ssem1)

    def sup(u, carry):
        rowbase = s * NBLK + u * SBI
        pltpu.sync_copy(src_hbm.at[pl.ds(rowbase, SBI)], sidx)
        pltpu.sync_copy(dstl_hbm.at[pl.ds(c * NROW + rowbase, SBI)], didx)
        gh = {}
        sh = {}
        gh[0] = pltpu.async_copy(g_hbm.at[sidx.at[0]], rows[0], gsem[0])
        for k in range(SBI):
            if k + 1 < SBI:
                b = (k + 1) % 2
                if k - 1 >= 0:
                    sh[k - 1].wait()
                gh[k + 1] = pltpu.async_copy(g_hbm.at[sidx.at[k + 1]],
                                             rows[b], gsem[b])
            gh[k].wait()
            sh[k] = pltpu.async_copy(rows[k % 2], agg.at[didx.at[k]],
                                     ssem[k % 2], add=True)
        sh[SBI - 2].wait()
        sh[SBI - 1].wait()
        return carry

    lax.fori_loop(0, NSUP, sup, 0)

    plsc.subcore_barrier()
    pltpu.sync_copy(agg.at[pl.ds(s * ORPT, ORPT)],
                    out_hbm.at[pl.ds(c * HROWS + s * ORPT, ORPT)])


@functools.cache
def _build_prop():
    return pl.kernel(
        _prop_body,
        out_type=jax.ShapeDtypeStruct((PN, D), jnp.float32),
        mesh=plsc.VectorSubcoreMesh(core_axis_name="c", subcore_axis_name="s"),
        scratch_types=[
            pltpu.VMEM_SHARED((HROWS, D), jnp.float32),
            pltpu.VMEM((SBI, EBLK), jnp.int32),
            pltpu.VMEM((SBI, EBLK), jnp.int32),
            pltpu.VMEM((EBLK, D), jnp.float32),
            pltpu.VMEM((EBLK, D), jnp.float32),
            pltpu.SemaphoreType.DMA,
            pltpu.SemaphoreType.DMA,
            pltpu.SemaphoreType.DMA,
            pltpu.SemaphoreType.DMA,
        ],
        compiler_params=pltpu.CompilerParams(use_tc_tiling_on_sc=False),
    )


def _prop(g, srcr, dstl, zeros_blk):
    return _build_prop()(g, srcr, dstl, zeros_blk)


MBLK = 1000


def _mlp_body(f_ref, w0_ref, b0_ref, w1_ref, b1_ref, deg_ref,
              h0_ref, g0_ref, sag_ref, sah_ref):
    f = f_ref[...]
    h = jnp.maximum(
        jnp.dot(f, w0_ref[...], preferred_element_type=jnp.float32)
        + b0_ref[...], 0.0)
    h0 = jnp.dot(h, w1_ref[...], preferred_element_type=jnp.float32) + b1_ref[...]
    deg = deg_ref[...][:, 0:1]
    norm = jnp.where(deg > 0, lax.rsqrt(jnp.maximum(deg, 1.0)), 0.0)
    h0_ref[...] = h0
    g0_ref[...] = h0 * norm
    sag_ref[...] = jnp.broadcast_to((1.0 - ALPHA) * norm * norm, h0.shape)
    sah_ref[...] = jnp.broadcast_to((1.0 - ALPHA) * norm, h0.shape)


def _mlp(features, W0, b0, W1, b1, deg40):
    out = jax.ShapeDtypeStruct((N, D), jnp.float32)
    return pl.pallas_call(
        _mlp_body,
        grid=(N // MBLK,),
        in_specs=[
            pl.BlockSpec((MBLK, 128), lambda i: (i, 0)),
            pl.BlockSpec((128, 64), lambda i: (0, 0)),
            pl.BlockSpec((1, 64), lambda i: (0, 0)),
            pl.BlockSpec((64, D), lambda i: (0, 0)),
            pl.BlockSpec((1, D), lambda i: (0, 0)),
            pl.BlockSpec((MBLK, D), lambda i: (i, 0)),
        ],
        out_specs=[pl.BlockSpec((MBLK, D), lambda i: (i, 0))] * 4,
        out_shape=[out, out, out, out],
    )(features, W0, b0, W1, b1, deg40)


UBLK = 256


def _upd_body(agg_ref, sa_ref, base_ref, out_ref):
    out_ref[...] = sa_ref[...] * agg_ref[...] + ALPHA * base_ref[...]


def _upd(agg, sa, base):
    return pl.pallas_call(
        _upd_body,
        grid=(PN // UBLK,),
        in_specs=[pl.BlockSpec((UBLK, D), lambda i: (i, 0))] * 3,
        out_specs=pl.BlockSpec((UBLK, D), lambda i: (i, 0)),
        out_shape=jax.ShapeDtypeStruct((PN, D), jnp.float32),
    )(agg, sa, base)


def _pad_rows(x):
    return jnp.pad(x.reshape(2, HALF, D), ((0, 0), (0, PAD), (0, 0))).reshape(PN, D)


def _unpad_rows(x):
    return x.reshape(2, HROWS, D)[:, :HALF].reshape(N, D)


def kernel(features, edge_index, W0, b0, W1, b1):
    src = edge_index[0].astype(jnp.int32)
    dst = edge_index[1].astype(jnp.int32)
    srcp = jnp.concatenate([src, jnp.zeros((EP - E,), jnp.int32)])
    srcr = jnp.where(srcp >= HALF, srcp + PAD, srcp).reshape(NROW, EBLK)
    dstp = jnp.concatenate([dst, jnp.full((EP - E,), -1, jnp.int32)])
    halves = []
    for cc in (0, 1):
        l = dstp - cc * HALF
        halves.append(jnp.where((l >= 0) & (l < HALF), l, TRASH))
    dstl = jnp.concatenate(halves).reshape(2 * NROW, EBLK)
    zeros_blk = jnp.zeros((ORPT, D), jnp.float32)

    ones_t = jnp.ones((PN, D), jnp.float32)
    deg40 = _unpad_rows(_prop(ones_t, srcr, dstl, zeros_blk))

    h0, g0, sag, sah = _mlp(features, W0, b0.reshape(1, -1), W1,
                            b1.reshape(1, -1), deg40)
    h0p = _pad_rows(h0)
    g0p = _pad_rows(g0)
    sagp = _pad_rows(sag)
    sahp = _pad_rows(sah)

    g = g0p
    for _ in range(K - 1):
        u = _prop(g, srcr, dstl, zeros_blk)
        g = _upd(u, sagp, g0p)
    u = _prop(g, srcr, dstl, zeros_blk)
    return _unpad_rows(_upd(u, sahp, h0p))
